# initial kernel scaffold (unmeasured)
import jax
import jax.numpy as jnp
from jax import lax
from jax.experimental import pallas as pl
from jax.experimental.pallas import tpu as pltpu

N_DEV = 4
SQ = 2048
D_MODEL = 1024
H_PER = 8
DH = 128
BLK = 64
N_RES = 4
BLKS_PER_RES = SQ // BLK // N_RES
GROUP = BLKS_PER_RES * BLK
SCALE = 0.08838834764831843


def kernel(x, Wq, K_ext, V_ext, Wo):
    my = lax.axis_index("i")
    Wq_my = lax.dynamic_slice(Wq, (0, my * H_PER * DH), (D_MODEL, H_PER * DH))
    Wo_my = lax.dynamic_slice(Wo, (my * H_PER * DH, 0), (H_PER * DH, D_MODEL))
    x2 = x[0]
    k2 = K_ext[0]
    v2 = V_ext[0]

    def body(x_ref, wq_ref, k_ref, v_ref, wo_ref, out_ref,
             comm_ref, send_sems, recv_sems):
        my_pos = lax.axis_index("i")
        left = lax.rem(my_pos + N_DEV - 1, N_DEV)
        right = lax.rem(my_pos + 1, N_DEV)

        barrier_sem = pltpu.get_barrier_semaphore()
        for nbr in (left, right):
            pl.semaphore_signal(
                barrier_sem, inc=1,
                device_id=(nbr,), device_id_type=pl.DeviceIdType.MESH,
            )
        pl.semaphore_wait(barrier_sem, 2)

        xv = x_ref[...]
        acc = jnp.zeros((SQ, D_MODEL), jnp.float32)
        for h in range(H_PER):
            q_h = jnp.dot(xv, wq_ref[:, h * DH:(h + 1) * DH],
                          preferred_element_type=jnp.float32)
            k_h = k_ref[:, h, :]
            v_h = v_ref[:, h, :]
            qg = q_h.reshape(BLKS_PER_RES, N_RES, BLK, DH)
            kg = k_h.reshape(BLKS_PER_RES, N_RES, BLK, DH)
            vg = v_h.reshape(BLKS_PER_RES, N_RES, BLK, DH)
            parts = []
            for r in range(N_RES):
                qr = qg[:, r].reshape(GROUP, DH)
                kr = kg[:, r].reshape(GROUP, DH)
                vr = vg[:, r].reshape(GROUP, DH)
                s = jnp.dot(qr, kr.T,
                            preferred_element_type=jnp.float32) * SCALE
                m = jnp.max(s, axis=-1, keepdims=True)
                w = jnp.exp(s - m)
                w = w / jnp.sum(w, axis=-1, keepdims=True)
                parts.append(
                    jnp.dot(w, vr, preferred_element_type=jnp.float32)
                    .reshape(BLKS_PER_RES, BLK, DH)
                )
            ctx_h = jnp.stack(parts, axis=1).reshape(SQ, DH)
            acc = acc + jnp.dot(ctx_h, wo_ref[h * DH:(h + 1) * DH, :],
                                preferred_element_type=jnp.float32)
        out_ref[...] = acc
        comm_ref[0] = acc

        for hop in range(N_DEV - 1):
            src = hop
            dst = (hop + 1) % 3
            rdma = pltpu.make_async_remote_copy(
                src_ref=comm_ref.at[src],
                dst_ref=comm_ref.at[dst],
                send_sem=send_sems.at[hop],
                recv_sem=recv_sems.at[hop],
                device_id=(right,),
                device_id_type=pl.DeviceIdType.MESH,
            )
            rdma.start()
            rdma.wait()
            out_ref[...] = out_ref[...] + comm_ref[dst]

    out = pl.pallas_call(
        body,
        out_shape=jax.ShapeDtypeStruct((SQ, D_MODEL), jnp.float32),
        in_specs=[pl.BlockSpec(memory_space=pltpu.VMEM)] * 5,
        out_specs=pl.BlockSpec(memory_space=pltpu.VMEM),
        scratch_shapes=[
            pltpu.VMEM((3, SQ, D_MODEL), jnp.float32),
            pltpu.SemaphoreType.DMA((N_DEV - 1,)),
            pltpu.SemaphoreType.DMA((N_DEV - 1,)),
        ],
        compiler_params=pltpu.CompilerParams(collective_id=0),
    )(x2, Wq_my, k2, v2, Wo_my)
    return out[None]


# baseline (device time: 370631 ns/iter reference)
import jax
import jax.numpy as jnp
from jax import lax
from jax.experimental import pallas as pl
from jax.experimental.pallas import tpu as pltpu

N_DEV = 4
SQ = 2048
D_MODEL = 1024
H_PER = 8
DH = 128
BLK = 64
N_RES = 4
BLKS_PER_RES = SQ // BLK // N_RES
GROUP = BLKS_PER_RES * BLK
SCALE = 0.08838834764831843


def _compute_body(x_ref, wq_ref, k_ref, v_ref, wo_ref, out_ref):
    xv = x_ref[...]
    out_ref[...] = jnp.zeros((SQ, D_MODEL), jnp.float32)
    for h in range(H_PER):
        q_h = jnp.dot(xv, wq_ref[:, h * DH:(h + 1) * DH],
                      preferred_element_type=jnp.float32)
        k_h = k_ref[:, h, :]
        v_h = v_ref[:, h, :]
        qg = q_h.reshape(BLKS_PER_RES, N_RES, BLK, DH)
        kg = k_h.reshape(BLKS_PER_RES, N_RES, BLK, DH)
        vg = v_h.reshape(BLKS_PER_RES, N_RES, BLK, DH)
        parts = []
        for r in range(N_RES):
            qr = qg[:, r].reshape(GROUP, DH)
            kr = kg[:, r].reshape(GROUP, DH)
            vr = vg[:, r].reshape(GROUP, DH)
            s = jnp.dot(qr, kr.T, preferred_element_type=jnp.float32) * SCALE
            m = jnp.max(s, axis=-1, keepdims=True)
            w = jnp.exp(s - m)
            w = w / jnp.sum(w, axis=-1, keepdims=True)
            parts.append(
                jnp.dot(w, vr, preferred_element_type=jnp.float32)
                .reshape(BLKS_PER_RES, BLK, DH)
            )
        ctx_h = jnp.stack(parts, axis=1).reshape(SQ, DH)
        out_ref[...] = out_ref[...] + jnp.dot(
            ctx_h, wo_ref[h * DH:(h + 1) * DH, :],
            preferred_element_type=jnp.float32)


def _allreduce_body(p_ref, out_ref, comm_ref, send_sems, recv_sems):
    my_pos = lax.axis_index("i")
    left = lax.rem(my_pos + N_DEV - 1, N_DEV)
    right = lax.rem(my_pos + 1, N_DEV)

    barrier_sem = pltpu.get_barrier_semaphore()
    for nbr in (left, right):
        pl.semaphore_signal(
            barrier_sem, inc=1,
            device_id=(nbr,), device_id_type=pl.DeviceIdType.MESH,
        )
    pl.semaphore_wait(barrier_sem, 2)

    comm_ref[0] = p_ref[...]
    out_ref[...] = p_ref[...]
    for hop in range(N_DEV - 1):
        src = hop
        dst = (hop + 1) % 3
        rdma = pltpu.make_async_remote_copy(
            src_ref=comm_ref.at[src],
            dst_ref=comm_ref.at[dst],
            send_sem=send_sems.at[hop],
            recv_sem=recv_sems.at[hop],
            device_id=(right,),
            device_id_type=pl.DeviceIdType.MESH,
        )
        rdma.start()
        rdma.wait()
        out_ref[...] = out_ref[...] + comm_ref[dst]


def kernel(x, Wq, K_ext, V_ext, Wo):
    my = lax.axis_index("i")
    Wq_my = lax.dynamic_slice(Wq, (0, my * H_PER * DH), (D_MODEL, H_PER * DH))
    Wo_my = lax.dynamic_slice(Wo, (my * H_PER * DH, 0), (H_PER * DH, D_MODEL))

    partial = pl.pallas_call(
        _compute_body,
        out_shape=jax.ShapeDtypeStruct((SQ, D_MODEL), jnp.float32),
        in_specs=[pl.BlockSpec(memory_space=pltpu.VMEM)] * 5,
        out_specs=pl.BlockSpec(memory_space=pltpu.VMEM),
        compiler_params=pltpu.CompilerParams(
            vmem_limit_bytes=60 * 1024 * 1024,
        ),
    )(x[0], Wq_my, K_ext[0], V_ext[0], Wo_my)

    out = pl.pallas_call(
        _allreduce_body,
        out_shape=jax.ShapeDtypeStruct((SQ, D_MODEL), jnp.float32),
        in_specs=[pl.BlockSpec(memory_space=pltpu.VMEM)],
        out_specs=pl.BlockSpec(memory_space=pltpu.VMEM),
        scratch_shapes=[
            pltpu.VMEM((3, SQ, D_MODEL), jnp.float32),
            pltpu.SemaphoreType.DMA((N_DEV - 1,)),
            pltpu.SemaphoreType.DMA((N_DEV - 1,)),
        ],
        compiler_params=pltpu.CompilerParams(
            collective_id=0,
            vmem_limit_bytes=60 * 1024 * 1024,
        ),
    )(partial)
    return out[None]


# device time: 238812 ns/iter; 1.5520x vs baseline; 1.5520x over previous
import jax
import jax.numpy as jnp
from jax import lax
from jax.experimental import pallas as pl
from jax.experimental.pallas import tpu as pltpu

N_DEV = 4
SQ = 2048
D_MODEL = 1024
H_PER = 8
DH = 128
BLK = 64
N_RES = 4
BLKS_PER_RES = SQ // BLK // N_RES
GROUP = BLKS_PER_RES * BLK
SCALE = 0.08838834764831843


def _compute_body(x_ref, wq_ref, k_ref, v_ref, wo_ref, out_ref):
    xv = x_ref[...]
    out_ref[...] = jnp.zeros((SQ, D_MODEL), jnp.float32)
    for h in range(H_PER):
        q_h = jnp.dot(xv, wq_ref[:, h * DH:(h + 1) * DH],
                      preferred_element_type=jnp.float32)
        k_h = k_ref[:, h, :]
        v_h = v_ref[:, h, :]
        qg = q_h.reshape(BLKS_PER_RES, N_RES, BLK, DH)
        kg = k_h.reshape(BLKS_PER_RES, N_RES, BLK, DH)
        vg = v_h.reshape(BLKS_PER_RES, N_RES, BLK, DH)
        parts = []
        for r in range(N_RES):
            qr = qg[:, r].reshape(GROUP, DH)
            kr = kg[:, r].reshape(GROUP, DH)
            vr = vg[:, r].reshape(GROUP, DH)
            s = jnp.dot(qr, kr.T, preferred_element_type=jnp.float32) * SCALE
            m = jnp.max(s, axis=-1, keepdims=True)
            w = jnp.exp(s - m)
            w = w / jnp.sum(w, axis=-1, keepdims=True)
            parts.append(
                jnp.dot(w, vr, preferred_element_type=jnp.float32)
                .reshape(BLKS_PER_RES, BLK, DH)
            )
        ctx_h = jnp.stack(parts, axis=1).reshape(SQ, DH)
        out_ref[...] = out_ref[...] + jnp.dot(
            ctx_h, wo_ref[h * DH:(h + 1) * DH, :],
            preferred_element_type=jnp.float32)


CHUNK = SQ // N_DEV


def _allreduce_body(p_ref, out_ref, rs_buf, send_sems, recv_sems):
    my_pos = lax.axis_index("i")
    left = lax.rem(my_pos + N_DEV - 1, N_DEV)
    right = lax.rem(my_pos + 1, N_DEV)

    barrier_sem = pltpu.get_barrier_semaphore()
    for nbr in (left, right):
        pl.semaphore_signal(
            barrier_sem, inc=1,
            device_id=(nbr,), device_id_type=pl.DeviceIdType.MESH,
        )
    pl.semaphore_wait(barrier_sem, 2)

    out_ref[...] = p_ref[...]

    for s in range(N_DEV - 1):
        sc = lax.rem(my_pos + 3 - s, N_DEV)
        rc = lax.rem(my_pos + 2 - s, N_DEV)
        rdma = pltpu.make_async_remote_copy(
            src_ref=out_ref.at[pl.ds(sc * CHUNK, CHUNK), :],
            dst_ref=rs_buf.at[s],
            send_sem=send_sems.at[s],
            recv_sem=recv_sems.at[s],
            device_id=(right,),
            device_id_type=pl.DeviceIdType.MESH,
        )
        rdma.start()
        rdma.wait()
        out_ref[pl.ds(rc * CHUNK, CHUNK), :] = (
            out_ref[pl.ds(rc * CHUNK, CHUNK), :] + rs_buf[s]
        )

    for t in range(N_DEV - 1):
        gc = lax.rem(my_pos + N_DEV - t, N_DEV)
        rdma = pltpu.make_async_remote_copy(
            src_ref=out_ref.at[pl.ds(gc * CHUNK, CHUNK), :],
            dst_ref=out_ref.at[pl.ds(gc * CHUNK, CHUNK), :],
            send_sem=send_sems.at[N_DEV - 1 + t],
            recv_sem=recv_sems.at[N_DEV - 1 + t],
            device_id=(right,),
            device_id_type=pl.DeviceIdType.MESH,
        )
        rdma.start()
        rdma.wait()


def kernel(x, Wq, K_ext, V_ext, Wo):
    my = lax.axis_index("i")
    Wq_my = lax.dynamic_slice(Wq, (0, my * H_PER * DH), (D_MODEL, H_PER * DH))
    Wo_my = lax.dynamic_slice(Wo, (my * H_PER * DH, 0), (H_PER * DH, D_MODEL))

    partial = pl.pallas_call(
        _compute_body,
        out_shape=jax.ShapeDtypeStruct((SQ, D_MODEL), jnp.float32),
        in_specs=[pl.BlockSpec(memory_space=pltpu.VMEM)] * 5,
        out_specs=pl.BlockSpec(memory_space=pltpu.VMEM),
        compiler_params=pltpu.CompilerParams(
            vmem_limit_bytes=60 * 1024 * 1024,
        ),
    )(x[0], Wq_my, K_ext[0], V_ext[0], Wo_my)

    out = pl.pallas_call(
        _allreduce_body,
        out_shape=jax.ShapeDtypeStruct((SQ, D_MODEL), jnp.float32),
        in_specs=[pl.BlockSpec(memory_space=pltpu.VMEM)],
        out_specs=pl.BlockSpec(memory_space=pltpu.VMEM),
        scratch_shapes=[
            pltpu.VMEM((N_DEV - 1, CHUNK, D_MODEL), jnp.float32),
            pltpu.SemaphoreType.DMA((2 * (N_DEV - 1),)),
            pltpu.SemaphoreType.DMA((2 * (N_DEV - 1),)),
        ],
        compiler_params=pltpu.CompilerParams(
            collective_id=0,
            vmem_limit_bytes=60 * 1024 * 1024,
        ),
    )(partial)
    return out[None]


# device time: 171583 ns/iter; 2.1601x vs baseline; 1.3918x over previous
import jax
import jax.numpy as jnp
from jax import lax
from jax.experimental import pallas as pl
from jax.experimental.pallas import tpu as pltpu

N_DEV = 4
SQ = 2048
D_MODEL = 1024
H_PER = 8
DH = 128
BLK = 64
N_RES = 4
BLKS_PER_RES = SQ // BLK // N_RES
GROUP = BLKS_PER_RES * BLK
SCALE = 0.08838834764831843


def _compute_body(x_ref, wq_ref, k_ref, v_ref, wo_ref, out_ref):
    xv = x_ref[...]
    out_ref[...] = jnp.zeros((SQ, D_MODEL), jnp.float32)
    for h in range(H_PER):
        q_h = jnp.dot(xv, wq_ref[:, h * DH:(h + 1) * DH],
                      preferred_element_type=jnp.float32)
        k_h = k_ref[:, h, :]
        v_h = v_ref[:, h, :]
        qg = q_h.reshape(BLKS_PER_RES, N_RES, BLK, DH)
        kg = k_h.reshape(BLKS_PER_RES, N_RES, BLK, DH)
        vg = v_h.reshape(BLKS_PER_RES, N_RES, BLK, DH)
        parts = []
        for r in range(N_RES):
            qr = qg[:, r].reshape(GROUP, DH)
            kr = kg[:, r].reshape(GROUP, DH)
            vr = vg[:, r].reshape(GROUP, DH)
            s = jnp.dot(qr, kr.T, preferred_element_type=jnp.float32) * SCALE
            m = jnp.max(s, axis=-1, keepdims=True)
            w = jnp.exp(s - m)
            w = w / jnp.sum(w, axis=-1, keepdims=True)
            parts.append(
                jnp.dot(w, vr, preferred_element_type=jnp.float32)
                .reshape(BLKS_PER_RES, BLK, DH)
            )
        ctx_h = jnp.stack(parts, axis=1).reshape(SQ, DH)
        out_ref[...] = out_ref[...] + jnp.dot(
            ctx_h, wo_ref[h * DH:(h + 1) * DH, :],
            preferred_element_type=jnp.float32)


CHUNK = SQ // N_DEV
HALF = D_MODEL // 2


def _allreduce_body(p_ref, out_ref, rs_r, rs_l, send_sems, recv_sems):
    my_pos = lax.axis_index("i")
    left = lax.rem(my_pos + N_DEV - 1, N_DEV)
    right = lax.rem(my_pos + 1, N_DEV)

    barrier_sem = pltpu.get_barrier_semaphore()
    for nbr in (left, right):
        pl.semaphore_signal(
            barrier_sem, inc=1,
            device_id=(nbr,), device_id_type=pl.DeviceIdType.MESH,
        )
    pl.semaphore_wait(barrier_sem, 2)

    out_ref[...] = p_ref[...]

    for s in range(N_DEV - 1):
        sc_r = lax.rem(my_pos + 3 - s, N_DEV)
        rc_r = lax.rem(my_pos + 2 - s, N_DEV)
        sc_l = lax.rem(my_pos + 1 + s, N_DEV)
        rc_l = lax.rem(my_pos + 2 + s, N_DEV)
        rdma_r = pltpu.make_async_remote_copy(
            src_ref=out_ref.at[pl.ds(sc_r * CHUNK, CHUNK), 0:HALF],
            dst_ref=rs_r.at[s],
            send_sem=send_sems.at[s],
            recv_sem=recv_sems.at[s],
            device_id=(right,),
            device_id_type=pl.DeviceIdType.MESH,
        )
        rdma_l = pltpu.make_async_remote_copy(
            src_ref=out_ref.at[pl.ds(sc_l * CHUNK, CHUNK), HALF:D_MODEL],
            dst_ref=rs_l.at[s],
            send_sem=send_sems.at[6 + s],
            recv_sem=recv_sems.at[6 + s],
            device_id=(left,),
            device_id_type=pl.DeviceIdType.MESH,
        )
        rdma_r.start()
        rdma_l.start()
        rdma_r.wait()
        rdma_l.wait()
        out_ref[pl.ds(rc_r * CHUNK, CHUNK), 0:HALF] = (
            out_ref[pl.ds(rc_r * CHUNK, CHUNK), 0:HALF] + rs_r[s]
        )
        out_ref[pl.ds(rc_l * CHUNK, CHUNK), HALF:D_MODEL] = (
            out_ref[pl.ds(rc_l * CHUNK, CHUNK), HALF:D_MODEL] + rs_l[s]
        )

    for t in range(N_DEV - 1):
        gc_r = lax.rem(my_pos + N_DEV - t, N_DEV)
        gc_l = lax.rem(my_pos + t, N_DEV)
        rdma_r = pltpu.make_async_remote_copy(
            src_ref=out_ref.at[pl.ds(gc_r * CHUNK, CHUNK), 0:HALF],
            dst_ref=out_ref.at[pl.ds(gc_r * CHUNK, CHUNK), 0:HALF],
            send_sem=send_sems.at[N_DEV - 1 + t],
            recv_sem=recv_sems.at[N_DEV - 1 + t],
            device_id=(right,),
            device_id_type=pl.DeviceIdType.MESH,
        )
        rdma_l = pltpu.make_async_remote_copy(
            src_ref=out_ref.at[pl.ds(gc_l * CHUNK, CHUNK), HALF:D_MODEL],
            dst_ref=out_ref.at[pl.ds(gc_l * CHUNK, CHUNK), HALF:D_MODEL],
            send_sem=send_sems.at[6 + N_DEV - 1 + t],
            recv_sem=recv_sems.at[6 + N_DEV - 1 + t],
            device_id=(left,),
            device_id_type=pl.DeviceIdType.MESH,
        )
        rdma_r.start()
        rdma_l.start()
        rdma_r.wait()
        rdma_l.wait()


def kernel(x, Wq, K_ext, V_ext, Wo):
    my = lax.axis_index("i")
    Wq_my = lax.dynamic_slice(Wq, (0, my * H_PER * DH), (D_MODEL, H_PER * DH))
    Wo_my = lax.dynamic_slice(Wo, (my * H_PER * DH, 0), (H_PER * DH, D_MODEL))

    partial = pl.pallas_call(
        _compute_body,
        out_shape=jax.ShapeDtypeStruct((SQ, D_MODEL), jnp.float32),
        in_specs=[pl.BlockSpec(memory_space=pltpu.VMEM)] * 5,
        out_specs=pl.BlockSpec(memory_space=pltpu.VMEM),
        compiler_params=pltpu.CompilerParams(
            vmem_limit_bytes=60 * 1024 * 1024,
        ),
    )(x[0], Wq_my, K_ext[0], V_ext[0], Wo_my)

    out = pl.pallas_call(
        _allreduce_body,
        out_shape=jax.ShapeDtypeStruct((SQ, D_MODEL), jnp.float32),
        in_specs=[pl.BlockSpec(memory_space=pltpu.VMEM)],
        out_specs=pl.BlockSpec(memory_space=pltpu.VMEM),
        scratch_shapes=[
            pltpu.VMEM((N_DEV - 1, CHUNK, HALF), jnp.float32),
            pltpu.VMEM((N_DEV - 1, CHUNK, HALF), jnp.float32),
            pltpu.SemaphoreType.DMA((12,)),
            pltpu.SemaphoreType.DMA((12,)),
        ],
        compiler_params=pltpu.CompilerParams(
            collective_id=0,
            vmem_limit_bytes=60 * 1024 * 1024,
        ),
    )(partial)
    return out[None]


# device time: 142154 ns/iter; 2.6072x vs baseline; 1.2070x over previous
import jax
import jax.numpy as jnp
from jax import lax
from jax.experimental import pallas as pl
from jax.experimental.pallas import tpu as pltpu

N_DEV = 4
SQ = 2048
D_MODEL = 1024
H_PER = 8
DH = 128
BLK = 64
N_RES = 4
BLKS_PER_RES = SQ // BLK // N_RES
GROUP = BLKS_PER_RES * BLK
SCALE = 0.08838834764831843


def _compute_body(x_ref, wq_ref, k_ref, v_ref, wo_ref, out_ref, acc_ref):
    xv = x_ref[...].astype(jnp.bfloat16)
    acc_ref[...] = jnp.zeros((SQ, D_MODEL), jnp.float32)
    for h in range(H_PER):
        wq_h = wq_ref[:, h * DH:(h + 1) * DH].astype(jnp.bfloat16)
        q_h = jnp.dot(xv, wq_h,
                      preferred_element_type=jnp.float32)
        k_h = k_ref[:, h, :].astype(jnp.bfloat16)
        v_h = v_ref[:, h, :].astype(jnp.bfloat16)
        qg = q_h.astype(jnp.bfloat16).reshape(BLKS_PER_RES, N_RES, BLK, DH)
        kg = k_h.reshape(BLKS_PER_RES, N_RES, BLK, DH)
        vg = v_h.reshape(BLKS_PER_RES, N_RES, BLK, DH)
        parts = []
        for r in range(N_RES):
            qr = qg[:, r].reshape(GROUP, DH)
            kr = kg[:, r].reshape(GROUP, DH)
            vr = vg[:, r].reshape(GROUP, DH)
            s = jnp.dot(qr, kr.T, preferred_element_type=jnp.float32) * SCALE
            m = jnp.max(s, axis=-1, keepdims=True)
            w = jnp.exp(s - m)
            w = (w / jnp.sum(w, axis=-1, keepdims=True)).astype(jnp.bfloat16)
            parts.append(
                jnp.dot(w, vr, preferred_element_type=jnp.float32)
                .reshape(BLKS_PER_RES, BLK, DH)
            )
        ctx_h = jnp.stack(parts, axis=1).reshape(SQ, DH).astype(jnp.bfloat16)
        wo_h = wo_ref[h * DH:(h + 1) * DH, :].astype(jnp.bfloat16)
        acc_ref[...] = acc_ref[...] + jnp.dot(
            ctx_h, wo_h, preferred_element_type=jnp.float32)
    out_ref[...] = acc_ref[...].astype(jnp.bfloat16)


CHUNK = SQ // N_DEV
HALF = D_MODEL // 2


def _allreduce_body(p_ref, out_ref, w_ref, rs_r, rs_l, send_sems, recv_sems):
    my_pos = lax.axis_index("i")
    left = lax.rem(my_pos + N_DEV - 1, N_DEV)
    right = lax.rem(my_pos + 1, N_DEV)

    barrier_sem = pltpu.get_barrier_semaphore()
    for nbr in (left, right):
        pl.semaphore_signal(
            barrier_sem, inc=1,
            device_id=(nbr,), device_id_type=pl.DeviceIdType.MESH,
        )
    pl.semaphore_wait(barrier_sem, 2)

    w_ref[...] = p_ref[...]

    for s in range(N_DEV - 1):
        sc_r = lax.rem(my_pos + 3 - s, N_DEV)
        rc_r = lax.rem(my_pos + 2 - s, N_DEV)
        sc_l = lax.rem(my_pos + 1 + s, N_DEV)
        rc_l = lax.rem(my_pos + 2 + s, N_DEV)
        rdma_r = pltpu.make_async_remote_copy(
            src_ref=w_ref.at[pl.ds(sc_r * CHUNK, CHUNK), 0:HALF],
            dst_ref=rs_r.at[s],
            send_sem=send_sems.at[s],
            recv_sem=recv_sems.at[s],
            device_id=(right,),
            device_id_type=pl.DeviceIdType.MESH,
        )
        rdma_l = pltpu.make_async_remote_copy(
            src_ref=w_ref.at[pl.ds(sc_l * CHUNK, CHUNK), HALF:D_MODEL],
            dst_ref=rs_l.at[s],
            send_sem=send_sems.at[6 + s],
            recv_sem=recv_sems.at[6 + s],
            device_id=(left,),
            device_id_type=pl.DeviceIdType.MESH,
        )
        rdma_r.start()
        rdma_l.start()
        rdma_r.wait()
        rdma_l.wait()
        w_ref[pl.ds(rc_r * CHUNK, CHUNK), 0:HALF] = (
            w_ref[pl.ds(rc_r * CHUNK, CHUNK), 0:HALF].astype(jnp.float32)
            + rs_r[s].astype(jnp.float32)
        ).astype(jnp.bfloat16)
        w_ref[pl.ds(rc_l * CHUNK, CHUNK), HALF:D_MODEL] = (
            w_ref[pl.ds(rc_l * CHUNK, CHUNK), HALF:D_MODEL].astype(jnp.float32)
            + rs_l[s].astype(jnp.float32)
        ).astype(jnp.bfloat16)

    for t in range(N_DEV - 1):
        gc_r = lax.rem(my_pos + N_DEV - t, N_DEV)
        gc_l = lax.rem(my_pos + t, N_DEV)
        rdma_r = pltpu.make_async_remote_copy(
            src_ref=w_ref.at[pl.ds(gc_r * CHUNK, CHUNK), 0:HALF],
            dst_ref=w_ref.at[pl.ds(gc_r * CHUNK, CHUNK), 0:HALF],
            send_sem=send_sems.at[N_DEV - 1 + t],
            recv_sem=recv_sems.at[N_DEV - 1 + t],
            device_id=(right,),
            device_id_type=pl.DeviceIdType.MESH,
        )
        rdma_l = pltpu.make_async_remote_copy(
            src_ref=w_ref.at[pl.ds(gc_l * CHUNK, CHUNK), HALF:D_MODEL],
            dst_ref=w_ref.at[pl.ds(gc_l * CHUNK, CHUNK), HALF:D_MODEL],
            send_sem=send_sems.at[6 + N_DEV - 1 + t],
            recv_sem=recv_sems.at[6 + N_DEV - 1 + t],
            device_id=(left,),
            device_id_type=pl.DeviceIdType.MESH,
        )
        rdma_r.start()
        rdma_l.start()
        rdma_r.wait()
        rdma_l.wait()
    out_ref[...] = w_ref[...].astype(jnp.float32)


def kernel(x, Wq, K_ext, V_ext, Wo):
    my = lax.axis_index("i")
    Wq_my = lax.dynamic_slice(Wq, (0, my * H_PER * DH), (D_MODEL, H_PER * DH))
    Wo_my = lax.dynamic_slice(Wo, (my * H_PER * DH, 0), (H_PER * DH, D_MODEL))

    partial = pl.pallas_call(
        _compute_body,
        out_shape=jax.ShapeDtypeStruct((SQ, D_MODEL), jnp.bfloat16),
        in_specs=[pl.BlockSpec(memory_space=pltpu.VMEM)] * 5,
        out_specs=pl.BlockSpec(memory_space=pltpu.VMEM),
        scratch_shapes=[
            pltpu.VMEM((SQ, D_MODEL), jnp.float32),
        ],
        compiler_params=pltpu.CompilerParams(
            vmem_limit_bytes=60 * 1024 * 1024,
        ),
    )(x[0], Wq_my, K_ext[0], V_ext[0], Wo_my)

    out = pl.pallas_call(
        _allreduce_body,
        out_shape=jax.ShapeDtypeStruct((SQ, D_MODEL), jnp.float32),
        in_specs=[pl.BlockSpec(memory_space=pltpu.VMEM)],
        out_specs=pl.BlockSpec(memory_space=pltpu.VMEM),
        scratch_shapes=[
            pltpu.VMEM((SQ, D_MODEL), jnp.bfloat16),
            pltpu.VMEM((N_DEV - 1, CHUNK, HALF), jnp.bfloat16),
            pltpu.VMEM((N_DEV - 1, CHUNK, HALF), jnp.bfloat16),
            pltpu.SemaphoreType.DMA((12,)),
            pltpu.SemaphoreType.DMA((12,)),
        ],
        compiler_params=pltpu.CompilerParams(
            collective_id=0,
            vmem_limit_bytes=60 * 1024 * 1024,
        ),
    )(partial)
    return out[None]
